# baseline (speedup 1.0000x reference)
import functools

import jax
import jax.numpy as jnp
from jax.experimental import pallas as pl
from jax.experimental.pallas import tpu as pltpu

_NHEAD = 4
_NH = 8
_EN = 2
_ALPHA = 0.2
_NB = 8


def _elu(r):
    return jnp.where(r > 0, r, jnp.exp(jnp.minimum(r, 0.0)) - 1.0)


def _gat_kernel(samples_blk, wall_ref, a_ref, wlast_ref, out_ref,
                wall_scr, mask_scr, r_scr, ebd_scr, aug_scr, h_scr, *, nb):
    f32, bf16 = jnp.float32, jnp.bfloat16
    n = wall_scr.shape[0]
    rows = n // nb
    i = pl.program_id(0)
    blk = samples_blk[0, 0]
    a_cat = a_ref[...]

    @pl.when(i < nb)
    def _build_wall():
        wall_scr[pl.ds(i * rows, rows), :] = jnp.dot(
            blk, wall_ref[...], preferred_element_type=f32)

    @pl.when(i == nb)
    def _build_vectors():
        w_all = wall_scr[...]
        aug_scr[:, :_NHEAD * _NH] = w_all.astype(bf16)
        aug_scr[:, _NHEAD * _NH:] = jnp.ones((n, 1), bf16)
        for k in range(_NHEAD):
            w_h = w_all[:, k * _NH:(k + 1) * _NH]
            f = jnp.sum(w_h * a_cat[k:k + 1, :], axis=1, keepdims=True)
            g = jax.lax.dot_general(
                a_cat[_NHEAD + k:_NHEAD + k + 1, :], w_h,
                dimension_numbers=(((1,), (1,)), ((), ())),
                preferred_element_type=f32)
            r_scr[:, k:k + 1] = jnp.exp((1.0 - _ALPHA) * f).astype(bf16)
            ebd_scr[k:k + 1, :] = jnp.exp(-g).astype(bf16)
            ebd_scr[_NHEAD + k:_NHEAD + k + 1, :] = jnp.exp(-_ALPHA * g).astype(bf16)

    @pl.when(i >= nb)
    def _attention_rows():
        b = i - nb
        mask_b = (blk != 0.0).astype(bf16)
        mask_scr[pl.ds(b * rows, rows), :] = mask_b
        aug = aug_scr[...]
        for k in range(_NHEAD):
            r_b = r_scr[pl.ds(b * rows, rows), k:k + 1]
            vals = jnp.minimum(r_b * ebd_scr[_NHEAD + k:_NHEAD + k + 1, :],
                               ebd_scr[k:k + 1, :]) * mask_b
            nd = jnp.dot(vals, aug, preferred_element_type=f32)
            h_scr[pl.ds(b * rows, rows), k * _NH:(k + 1) * _NH] = _elu(
                nd[:, k * _NH:(k + 1) * _NH] / nd[:, _NHEAD * _NH:])

    @pl.when(i == 2 * nb - 1)
    def _layer2():
        w2 = jnp.dot(h_scr[...], wlast_ref[...], preferred_element_type=f32)
        f2 = jnp.sum(w2 * a_cat[8:9, :_EN], axis=1, keepdims=True)
        g2 = jax.lax.dot_general(
            a_cat[9:10, :_EN], w2,
            dimension_numbers=(((1,), (1,)), ((), ())),
            preferred_element_type=f32)
        r2 = jnp.exp((1.0 - _ALPHA) * f2).astype(bf16)
        eb2 = jnp.exp(-g2).astype(bf16)
        ed2 = jnp.exp(-_ALPHA * g2).astype(bf16)
        vals2 = jnp.minimum(r2 * ed2, eb2) * mask_scr[...]
        aug2 = jnp.concatenate(
            [w2.astype(bf16), jnp.ones((n, 1), bf16)], axis=1)
        nd2 = jnp.dot(vals2, aug2, preferred_element_type=f32)
        out_ref[...] = _elu(nd2[:, :_EN] / nd2[:, _EN:_EN + 1])


def kernel(samples, W0, a0, W1, a1, W2, a2, W3, a3, W_last, a_last):
    f32, bf16 = jnp.float32, jnp.bfloat16
    n = samples.shape[2]
    rows = n // _NB
    w_all = jnp.concatenate([W0, W1, W2, W3], axis=1)
    heads_a = jnp.concatenate([a0, a1, a2, a3], axis=0)
    a_cat = jnp.zeros((16, _NH), f32)
    a_cat = a_cat.at[0:4, :].set(heads_a[:, :_NH])
    a_cat = a_cat.at[4:8, :].set(heads_a[:, _NH:])
    a_cat = a_cat.at[8, :_EN].set(a_last[0, :_EN])
    a_cat = a_cat.at[9, :_EN].set(a_last[0, _EN:])

    outs = []
    for s in range(samples.shape[0]):
        call = pl.pallas_call(
            functools.partial(_gat_kernel, nb=_NB),
            grid=(2 * _NB,),
            in_specs=[
                pl.BlockSpec((1, 1, rows, n),
                             lambda i, s=s: (s, i // _NB, i % _NB, 0)),
                pl.BlockSpec((n, _NHEAD * _NH), lambda i: (0, 0)),
                pl.BlockSpec((16, _NH), lambda i: (0, 0)),
                pl.BlockSpec((_NHEAD * _NH, _EN), lambda i: (0, 0)),
            ],
            out_specs=pl.BlockSpec((n, _EN), lambda i: (0, 0)),
            out_shape=jax.ShapeDtypeStruct((n, _EN), f32),
            scratch_shapes=[
                pltpu.VMEM((n, _NHEAD * _NH), f32),
                pltpu.VMEM((n, n), bf16),
                pltpu.VMEM((n, _NHEAD), bf16),
                pltpu.VMEM((2 * _NHEAD, n), bf16),
                pltpu.VMEM((n, _NHEAD * _NH + 1), bf16),
                pltpu.VMEM((n, _NHEAD * _NH), f32),
            ],
            compiler_params=pltpu.CompilerParams(
                vmem_limit_bytes=100 * 1024 * 1024),
        )
        outs.append(call(samples, w_all, a_cat, W_last))
    return jnp.stack(outs, 0)


# speedup vs baseline: 1.2075x; 1.2075x over previous
import functools

import jax
import jax.numpy as jnp
from jax.experimental import pallas as pl
from jax.experimental.pallas import tpu as pltpu

_NHEAD = 4
_NH = 8
_EN = 2
_ALPHA = 0.2
_NB = 4


def _elu(r):
    return jnp.where(r > 0, r, jnp.exp(jnp.minimum(r, 0.0)) - 1.0)


def _gat_kernel(samples_blk, wall_ref, a_ref, wlast_ref, out_ref,
                wall_scr, mask_scr, r_scr, ebd_scr, aug_scr, h_scr, *, nb):
    f32, bf16 = jnp.float32, jnp.bfloat16
    n = wall_scr.shape[0]
    rows = n // nb
    i = pl.program_id(0)
    blk = samples_blk[0, 0]
    a_cat = a_ref[...]

    @pl.when(i < nb)
    def _build_wall():
        wall_scr[pl.ds(i * rows, rows), :] = jnp.dot(
            blk, wall_ref[...], preferred_element_type=f32)

    @pl.when(i == nb)
    def _build_vectors():
        w_all = wall_scr[...]
        aug_scr[:, :_NHEAD * _NH] = w_all.astype(bf16)
        aug_scr[:, _NHEAD * _NH:] = jnp.ones((n, 1), bf16)
        for k in range(_NHEAD):
            w_h = w_all[:, k * _NH:(k + 1) * _NH]
            f = jnp.sum(w_h * a_cat[k:k + 1, :], axis=1, keepdims=True)
            g = jax.lax.dot_general(
                a_cat[_NHEAD + k:_NHEAD + k + 1, :], w_h,
                dimension_numbers=(((1,), (1,)), ((), ())),
                preferred_element_type=f32)
            r_scr[:, k:k + 1] = jnp.exp((1.0 - _ALPHA) * f).astype(bf16)
            ebd_scr[k:k + 1, :] = jnp.exp(-g).astype(bf16)
            ebd_scr[_NHEAD + k:_NHEAD + k + 1, :] = jnp.exp(-_ALPHA * g).astype(bf16)

    @pl.when(i >= nb)
    def _attention_rows():
        b = i - nb
        mask_b = (blk != 0.0).astype(bf16)
        mask_scr[pl.ds(b * rows, rows), :] = mask_b
        aug = aug_scr[...]
        for k in range(_NHEAD):
            r_b = r_scr[pl.ds(b * rows, rows), k:k + 1]
            vals = jnp.minimum(r_b * ebd_scr[_NHEAD + k:_NHEAD + k + 1, :],
                               ebd_scr[k:k + 1, :]) * mask_b
            nd = jnp.dot(vals, aug, preferred_element_type=f32)
            h_scr[pl.ds(b * rows, rows), k * _NH:(k + 1) * _NH] = _elu(
                nd[:, k * _NH:(k + 1) * _NH] / nd[:, _NHEAD * _NH:])

    @pl.when(i == 2 * nb - 1)
    def _layer2():
        w2 = jnp.dot(h_scr[...], wlast_ref[...], preferred_element_type=f32)
        f2 = jnp.sum(w2 * a_cat[8:9, :_EN], axis=1, keepdims=True)
        g2 = jax.lax.dot_general(
            a_cat[9:10, :_EN], w2,
            dimension_numbers=(((1,), (1,)), ((), ())),
            preferred_element_type=f32)
        r2 = jnp.exp((1.0 - _ALPHA) * f2).astype(bf16)
        eb2 = jnp.exp(-g2).astype(bf16)
        ed2 = jnp.exp(-_ALPHA * g2).astype(bf16)
        vals2 = jnp.minimum(r2 * ed2, eb2) * mask_scr[...]
        aug2 = jnp.concatenate(
            [w2.astype(bf16), jnp.ones((n, 1), bf16)], axis=1)
        nd2 = jnp.dot(vals2, aug2, preferred_element_type=f32)
        out_ref[...] = _elu(nd2[:, :_EN] / nd2[:, _EN:_EN + 1])


def kernel(samples, W0, a0, W1, a1, W2, a2, W3, a3, W_last, a_last):
    f32, bf16 = jnp.float32, jnp.bfloat16
    n = samples.shape[2]
    rows = n // _NB
    w_all = jnp.concatenate([W0, W1, W2, W3], axis=1)
    heads_a = jnp.concatenate([a0, a1, a2, a3], axis=0)
    a_cat = jnp.zeros((16, _NH), f32)
    a_cat = a_cat.at[0:4, :].set(heads_a[:, :_NH])
    a_cat = a_cat.at[4:8, :].set(heads_a[:, _NH:])
    a_cat = a_cat.at[8, :_EN].set(a_last[0, :_EN])
    a_cat = a_cat.at[9, :_EN].set(a_last[0, _EN:])

    outs = []
    for s in range(samples.shape[0]):
        call = pl.pallas_call(
            functools.partial(_gat_kernel, nb=_NB),
            grid=(2 * _NB,),
            in_specs=[
                pl.BlockSpec((1, 1, rows, n),
                             lambda i, s=s: (s, i // _NB, i % _NB, 0)),
                pl.BlockSpec((n, _NHEAD * _NH), lambda i: (0, 0)),
                pl.BlockSpec((16, _NH), lambda i: (0, 0)),
                pl.BlockSpec((_NHEAD * _NH, _EN), lambda i: (0, 0)),
            ],
            out_specs=pl.BlockSpec((n, _EN), lambda i: (0, 0)),
            out_shape=jax.ShapeDtypeStruct((n, _EN), f32),
            scratch_shapes=[
                pltpu.VMEM((n, _NHEAD * _NH), f32),
                pltpu.VMEM((n, n), bf16),
                pltpu.VMEM((n, _NHEAD), bf16),
                pltpu.VMEM((2 * _NHEAD, n), bf16),
                pltpu.VMEM((n, _NHEAD * _NH + 1), bf16),
                pltpu.VMEM((n, _NHEAD * _NH), f32),
            ],
            compiler_params=pltpu.CompilerParams(
                vmem_limit_bytes=100 * 1024 * 1024),
        )
        outs.append(call(samples, w_all, a_cat, W_last))
    return jnp.stack(outs, 0)


# speedup vs baseline: 1.3052x; 1.0809x over previous
import functools

import jax
import jax.numpy as jnp
from jax.experimental import pallas as pl
from jax.experimental.pallas import tpu as pltpu

_NHEAD = 4
_NH = 8
_EN = 2
_ALPHA = 0.2
_NB = 2


def _elu(r):
    return jnp.where(r > 0, r, jnp.exp(jnp.minimum(r, 0.0)) - 1.0)


def _gat_kernel(samples_blk, wall_ref, a_ref, wlast_ref, out_ref,
                wall_scr, mask_scr, r_scr, ebd_scr, aug_scr, h_scr, *, nb):
    f32, bf16 = jnp.float32, jnp.bfloat16
    n = wall_scr.shape[0]
    rows = n // nb
    i = pl.program_id(0)
    blk = samples_blk[0, 0]
    a_cat = a_ref[...]

    @pl.when(i < nb)
    def _build_wall():
        wall_scr[pl.ds(i * rows, rows), :] = jnp.dot(
            blk, wall_ref[...], preferred_element_type=f32)

    @pl.when(i == nb)
    def _build_vectors():
        w_all = wall_scr[...]
        aug_scr[:, :_NHEAD * _NH] = w_all.astype(bf16)
        aug_scr[:, _NHEAD * _NH:] = jnp.ones((n, 1), bf16)
        for k in range(_NHEAD):
            w_h = w_all[:, k * _NH:(k + 1) * _NH]
            f = jnp.sum(w_h * a_cat[k:k + 1, :], axis=1, keepdims=True)
            g = jax.lax.dot_general(
                a_cat[_NHEAD + k:_NHEAD + k + 1, :], w_h,
                dimension_numbers=(((1,), (1,)), ((), ())),
                preferred_element_type=f32)
            r_scr[:, k:k + 1] = jnp.exp((1.0 - _ALPHA) * f).astype(bf16)
            ebd_scr[k:k + 1, :] = jnp.exp(-g).astype(bf16)
            ebd_scr[_NHEAD + k:_NHEAD + k + 1, :] = jnp.exp(-_ALPHA * g).astype(bf16)

    @pl.when(i >= nb)
    def _attention_rows():
        b = i - nb
        mask_b = (blk != 0.0).astype(bf16)
        mask_scr[pl.ds(b * rows, rows), :] = mask_b
        aug = aug_scr[...]
        for k in range(_NHEAD):
            r_b = r_scr[pl.ds(b * rows, rows), k:k + 1]
            vals = jnp.minimum(r_b * ebd_scr[_NHEAD + k:_NHEAD + k + 1, :],
                               ebd_scr[k:k + 1, :]) * mask_b
            nd = jnp.dot(vals, aug, preferred_element_type=f32)
            h_scr[pl.ds(b * rows, rows), k * _NH:(k + 1) * _NH] = _elu(
                nd[:, k * _NH:(k + 1) * _NH] / nd[:, _NHEAD * _NH:])

    @pl.when(i == 2 * nb - 1)
    def _layer2():
        w2 = jnp.dot(h_scr[...], wlast_ref[...], preferred_element_type=f32)
        f2 = jnp.sum(w2 * a_cat[8:9, :_EN], axis=1, keepdims=True)
        g2 = jax.lax.dot_general(
            a_cat[9:10, :_EN], w2,
            dimension_numbers=(((1,), (1,)), ((), ())),
            preferred_element_type=f32)
        r2 = jnp.exp((1.0 - _ALPHA) * f2).astype(bf16)
        eb2 = jnp.exp(-g2).astype(bf16)
        ed2 = jnp.exp(-_ALPHA * g2).astype(bf16)
        vals2 = jnp.minimum(r2 * ed2, eb2) * mask_scr[...]
        aug2 = jnp.concatenate(
            [w2.astype(bf16), jnp.ones((n, 1), bf16)], axis=1)
        nd2 = jnp.dot(vals2, aug2, preferred_element_type=f32)
        out_ref[...] = _elu(nd2[:, :_EN] / nd2[:, _EN:_EN + 1])


def kernel(samples, W0, a0, W1, a1, W2, a2, W3, a3, W_last, a_last):
    f32, bf16 = jnp.float32, jnp.bfloat16
    n = samples.shape[2]
    rows = n // _NB
    w_all = jnp.concatenate([W0, W1, W2, W3], axis=1)
    heads_a = jnp.concatenate([a0, a1, a2, a3], axis=0)
    a_cat = jnp.zeros((16, _NH), f32)
    a_cat = a_cat.at[0:4, :].set(heads_a[:, :_NH])
    a_cat = a_cat.at[4:8, :].set(heads_a[:, _NH:])
    a_cat = a_cat.at[8, :_EN].set(a_last[0, :_EN])
    a_cat = a_cat.at[9, :_EN].set(a_last[0, _EN:])

    outs = []
    for s in range(samples.shape[0]):
        call = pl.pallas_call(
            functools.partial(_gat_kernel, nb=_NB),
            grid=(2 * _NB,),
            in_specs=[
                pl.BlockSpec((1, 1, rows, n),
                             lambda i, s=s: (s, i // _NB, i % _NB, 0)),
                pl.BlockSpec((n, _NHEAD * _NH), lambda i: (0, 0)),
                pl.BlockSpec((16, _NH), lambda i: (0, 0)),
                pl.BlockSpec((_NHEAD * _NH, _EN), lambda i: (0, 0)),
            ],
            out_specs=pl.BlockSpec((n, _EN), lambda i: (0, 0)),
            out_shape=jax.ShapeDtypeStruct((n, _EN), f32),
            scratch_shapes=[
                pltpu.VMEM((n, _NHEAD * _NH), f32),
                pltpu.VMEM((n, n), bf16),
                pltpu.VMEM((n, _NHEAD), bf16),
                pltpu.VMEM((2 * _NHEAD, n), bf16),
                pltpu.VMEM((n, _NHEAD * _NH + 1), bf16),
                pltpu.VMEM((n, _NHEAD * _NH), f32),
            ],
            compiler_params=pltpu.CompilerParams(
                vmem_limit_bytes=100 * 1024 * 1024),
        )
        outs.append(call(samples, w_all, a_cat, W_last))
    return jnp.stack(outs, 0)
